# baseline (speedup 1.0000x reference)
import functools

import jax
import jax.numpy as jnp
from jax import lax
from jax.experimental import pallas as pl
from jax.experimental.pallas import tpu as pltpu
from jax.experimental.pallas import tpu_sc as plsc

B, S, D, C = 4, 8192, 128, 8
NCOL = D * C
LANES = 16
KTOP = 8
CHUNK = 128
NCHUNK = S // CHUNK
STRIPE = 128
GSTRIPE = STRIPE // LANES

_SORT8 = [(0, 1), (2, 3), (4, 5), (6, 7), (0, 2), (1, 3), (4, 6), (5, 7),
          (1, 2), (5, 6), (0, 4), (1, 5), (2, 6), (3, 7), (2, 4), (3, 5),
          (1, 2), (3, 4), (5, 6)]
_BITONIC8 = [(0, 4), (1, 5), (2, 6), (3, 7), (0, 2), (1, 3), (4, 6), (5, 7),
             (0, 1), (2, 3), (4, 5), (6, 7)]


def _merge_batch(R, N):
    N = list(N)
    for i, j in _SORT8:
        hi = jnp.maximum(N[i], N[j])
        lo = jnp.minimum(N[i], N[j])
        N[i], N[j] = hi, lo
    return _merge_sorted(R, N)


def _merge_sorted(R, N):
    M = [jnp.maximum(R[i], N[KTOP - 1 - i]) for i in range(KTOP)]
    for i, j in _BITONIC8:
        hi = jnp.maximum(M[i], M[j])
        lo = jnp.minimum(M[i], M[j])
        M[i], M[j] = hi, lo
    return tuple(M)


NBUF = 4


def _sc_body(x_hbm, out_hbm, buf0, buf1, buf2, buf3, acc,
             sem0, sem1, sem2, sem3):
    info = plsc.get_sparse_core_info()
    nc = info.num_cores
    wid = lax.axis_index("s") * nc + lax.axis_index("c")
    bufs = (buf0, buf1, buf2, buf3)
    sems = (sem0, sem1, sem2, sem3)
    b = wid // (NCOL // STRIPE)
    col0 = (wid % (NCOL // STRIPE)) * STRIPE

    neg = jnp.full((LANES,), -jnp.inf, jnp.float32)
    for j in range(KTOP):
        for g in range(GSTRIPE):
            acc[j, pl.ds(g * LANES, LANES)] = neg

    def chunk_copy(c, slot):
        return pltpu.async_copy(
            x_hbm.at[b, pl.ds(c * CHUNK, CHUNK), pl.ds(col0, STRIPE)],
            bufs[slot], sems[slot])

    def group_pass(g, buf):
        off = pl.multiple_of(g * LANES, LANES)
        R = tuple(acc[j, pl.ds(off, LANES)] for j in range(KTOP))

        def body(k, R):
            rows = tuple(buf[k * KTOP + i, pl.ds(off, LANES)]
                         for i in range(KTOP))
            return _merge_batch(R, rows)

        R = lax.fori_loop(0, CHUNK // KTOP, body, R, unroll=2)
        for j in range(KTOP):
            acc[j, pl.ds(off, LANES)] = R[j]

    def chunk_step(c, slot):
        pltpu.make_async_copy(
            x_hbm.at[b, pl.ds(c * CHUNK, CHUNK), pl.ds(col0, STRIPE)],
            bufs[slot], sems[slot]).wait()

        @pl.when(c + NBUF - 1 < NCHUNK)
        def _():
            chunk_copy(c + NBUF - 1, (slot + NBUF - 1) % NBUF)

        lax.fori_loop(0, GSTRIPE,
                      lambda g, _: (group_pass(g, bufs[slot]), 0)[1], 0)

    for p in range(NBUF - 1):
        chunk_copy(p, p)

    def outer(cc, _):
        for sl in range(NBUF):
            chunk_step(cc * NBUF + sl, sl)
        return 0

    lax.fori_loop(0, NCHUNK // NBUF, outer, 0)
    pltpu.sync_copy(acc, out_hbm.at[b, :, pl.ds(col0, STRIPE)])


def kernel(inputs):
    x3 = inputs.reshape(B, S, NCOL)
    mesh = plsc.VectorSubcoreMesh(core_axis_name="c", subcore_axis_name="s")
    run = functools.partial(
        pl.kernel, mesh=mesh,
        out_type=jax.ShapeDtypeStruct((B, KTOP, NCOL), jnp.float32),
        scratch_types=[
            pltpu.VMEM((CHUNK, STRIPE), jnp.float32),
            pltpu.VMEM((CHUNK, STRIPE), jnp.float32),
            pltpu.VMEM((CHUNK, STRIPE), jnp.float32),
            pltpu.VMEM((CHUNK, STRIPE), jnp.float32),
            pltpu.VMEM((KTOP, STRIPE), jnp.float32),
            pltpu.SemaphoreType.DMA,
            pltpu.SemaphoreType.DMA,
            pltpu.SemaphoreType.DMA,
            pltpu.SemaphoreType.DMA,
        ],
    )(_sc_body)
    return run(x3).reshape(B, KTOP, D, C)


# speedup vs baseline: 1.0108x; 1.0108x over previous
import functools

import jax
import jax.numpy as jnp
from jax import lax
from jax.experimental import pallas as pl
from jax.experimental.pallas import tpu as pltpu
from jax.experimental.pallas import tpu_sc as plsc

B, S, D, C = 4, 8192, 128, 8
NCOL = D * C
LANES = 16
KTOP = 8
CHUNK = 256
NCHUNK = S // CHUNK
STRIPE = 128
GSTRIPE = STRIPE // LANES

_SORT8 = [(0, 1), (2, 3), (4, 5), (6, 7), (0, 2), (1, 3), (4, 6), (5, 7),
          (1, 2), (5, 6), (0, 4), (1, 5), (2, 6), (3, 7), (2, 4), (3, 5),
          (1, 2), (3, 4), (5, 6)]
_BITONIC8 = [(0, 4), (1, 5), (2, 6), (3, 7), (0, 2), (1, 3), (4, 6), (5, 7),
             (0, 1), (2, 3), (4, 5), (6, 7)]


def _merge_batch(R, N):
    N = list(N)
    for i, j in _SORT8:
        hi = jnp.maximum(N[i], N[j])
        lo = jnp.minimum(N[i], N[j])
        N[i], N[j] = hi, lo
    return _merge_sorted(R, N)


def _merge_sorted(R, N):
    M = [jnp.maximum(R[i], N[KTOP - 1 - i]) for i in range(KTOP)]
    for i, j in _BITONIC8:
        hi = jnp.maximum(M[i], M[j])
        lo = jnp.minimum(M[i], M[j])
        M[i], M[j] = hi, lo
    return tuple(M)


def _sc_body(x_hbm, out_hbm, buf0, buf1, acc, sem0, sem1):
    info = plsc.get_sparse_core_info()
    nc = info.num_cores
    wid = lax.axis_index("s") * nc + lax.axis_index("c")
    bufs = (buf0, buf1)
    sems = (sem0, sem1)
    b = wid // (NCOL // STRIPE)
    col0 = (wid % (NCOL // STRIPE)) * STRIPE

    neg = jnp.full((LANES,), -jnp.inf, jnp.float32)
    for j in range(KTOP):
        for g in range(GSTRIPE):
            acc[j, pl.ds(g * LANES, LANES)] = neg

    def chunk_copy(c, slot):
        return pltpu.async_copy(
            x_hbm.at[b, pl.ds(c * CHUNK, CHUNK), pl.ds(col0, STRIPE)],
            bufs[slot], sems[slot])

    def group_pass(g, buf):
        off = pl.multiple_of(g * LANES, LANES)
        R = tuple(acc[j, pl.ds(off, LANES)] for j in range(KTOP))

        def body(k, R):
            rows = tuple(buf[k * KTOP + i, pl.ds(off, LANES)]
                         for i in range(KTOP))
            return _merge_batch(R, rows)

        R = lax.fori_loop(0, CHUNK // KTOP, body, R, unroll=2)
        for j in range(KTOP):
            acc[j, pl.ds(off, LANES)] = R[j]

    def chunk_step(c, slot):
        pltpu.make_async_copy(
            x_hbm.at[b, pl.ds(c * CHUNK, CHUNK), pl.ds(col0, STRIPE)],
            bufs[slot], sems[slot]).wait()

        @pl.when(c + 1 < NCHUNK)
        def _():
            chunk_copy(c + 1, 1 - slot)

        lax.fori_loop(0, GSTRIPE,
                      lambda g, _: (group_pass(g, bufs[slot]), 0)[1], 0)

    chunk_copy(0, 0)

    def outer(cc, _):
        chunk_step(cc * 2, 0)
        chunk_step(cc * 2 + 1, 1)
        return 0

    lax.fori_loop(0, NCHUNK // 2, outer, 0)
    pltpu.sync_copy(acc, out_hbm.at[b, :, pl.ds(col0, STRIPE)])


def kernel(inputs):
    x3 = inputs.reshape(B, S, NCOL)
    mesh = plsc.VectorSubcoreMesh(core_axis_name="c", subcore_axis_name="s")
    run = functools.partial(
        pl.kernel, mesh=mesh,
        out_type=jax.ShapeDtypeStruct((B, KTOP, NCOL), jnp.float32),
        scratch_types=[
            pltpu.VMEM((CHUNK, STRIPE), jnp.float32),
            pltpu.VMEM((CHUNK, STRIPE), jnp.float32),
            pltpu.VMEM((KTOP, STRIPE), jnp.float32),
            pltpu.SemaphoreType.DMA,
            pltpu.SemaphoreType.DMA,
        ],
    )(_sc_body)
    return run(x3).reshape(B, KTOP, D, C)
